# Initial kernel scaffold; baseline (speedup 1.0000x reference)
import jax
import jax.numpy as jnp
from jax.experimental import pallas as pl


def kernel(points, features):
    raise NotImplementedError("write your pallas kernel here")


# speedup vs baseline: 214.7606x; 214.7606x over previous
import functools

import jax
import jax.numpy as jnp
from jax import lax
from jax.experimental import pallas as pl
from jax.experimental.pallas import tpu as pltpu

RADIUS2 = 0.2 * 0.2
NSAMPLE = 16


def _fps_kernel(px_ref, py_ref, pz_ref, sel_ref, *, nrow, ncol, npoint):
    n = nrow * ncol
    px = px_ref[0]
    py = py_ref[0]
    pz = pz_ref[0]
    row_iota = lax.broadcasted_iota(jnp.int32, (nrow, ncol), 0)
    col_iota = lax.broadcasted_iota(jnp.int32, (nrow, ncol), 1)
    flat_iota = row_iota * ncol + col_iota
    lane_iota = lax.broadcasted_iota(jnp.int32, (1, ncol), 1)

    def body(i, carry):
        dists, far = carry
        r = far // ncol
        c = far % ncol
        lm = lane_iota == c
        cx = jnp.sum(jnp.where(lm, px_ref[0, pl.ds(r, 1), :], 0.0))
        cy = jnp.sum(jnp.where(lm, py_ref[0, pl.ds(r, 1), :], 0.0))
        cz = jnp.sum(jnp.where(lm, pz_ref[0, pl.ds(r, 1), :], 0.0))
        sel_ref[0, 0, i] = cx
        sel_ref[0, 1, i] = cy
        sel_ref[0, 2, i] = cz
        dx = px - cx
        dy = py - cy
        dz = pz - cz
        d = dx * dx + dy * dy + dz * dz
        dists = jnp.minimum(dists, d)
        m = jnp.max(dists)
        far2 = jnp.min(jnp.where(dists == m, flat_iota, jnp.int32(n)))
        return dists, far2

    dists0 = jnp.full((nrow, ncol), 1e10, jnp.float32)
    lax.fori_loop(0, npoint, body, (dists0, jnp.int32(0)))


def _ball_pool_kernel(sx_ref, sy_ref, sz_ref, px_ref, py_ref, pz_ref,
                      feat_ref, out_ref, *, tc, nrow, ncol):
    n = nrow * ncol
    cx = sx_ref[0]
    cy = sy_ref[0]
    cz = sz_ref[0]
    pxr = px_ref[0]
    pyr = py_ref[0]
    pzr = pz_ref[0]
    dx = cx - pxr
    dy = cy - pyr
    dz = cz - pzr
    d2 = dx * dx + dy * dy + dz * dz
    w = (d2 < RADIUS2).astype(jnp.float32)
    w3 = w.reshape(tc, nrow, ncol)

    i0 = lax.broadcasted_iota(jnp.int32, (ncol, ncol), 0)
    i1 = lax.broadcasted_iota(jnp.int32, (ncol, ncol), 1)
    triu = (i0 <= i1).astype(jnp.float32)
    incl = lax.dot_general(w3, triu, (((2,), (0,)), ((), ())),
                           preferred_element_type=jnp.float32)
    blocktot = incl[:, :, ncol - 1]
    b0 = lax.broadcasted_iota(jnp.int32, (nrow, nrow), 0)
    b1 = lax.broadcasted_iota(jnp.int32, (nrow, nrow), 1)
    sup = (b0 < b1).astype(jnp.float32)
    offs = lax.dot_general(blocktot, sup, (((1,), (0,)), ((), ())),
                           preferred_element_type=jnp.float32)
    rank_ex = offs[:, :, None] + incl - w3

    w16 = w3 * (rank_ex < float(NSAMPLE)).astype(jnp.float32)
    first = w3 * (rank_ex == 0.0).astype(jnp.float32)
    total = offs[:, nrow - 1] + blocktot[:, nrow - 1]
    cnt16 = jnp.minimum(total, float(NSAMPLE))
    pw = w16 + first * (float(NSAMPLE) - cnt16)[:, None, None]
    zer = (lax.broadcasted_iota(jnp.int32, (nrow, ncol), 0) == 0) & (
        lax.broadcasted_iota(jnp.int32, (nrow, ncol), 1) == 0)
    pw = pw + (total == 0.0)[:, None, None] * zer[None].astype(jnp.float32) * float(NSAMPLE)

    pmat = pw.reshape(tc, n)
    pooled = lax.dot_general(pmat, feat_ref[0], (((1,), (0,)), ((), ())),
                             preferred_element_type=jnp.float32)
    out_ref[0] = pooled * (1.0 / NSAMPLE)


def kernel(points, features):
    B, N, C = features.shape
    stride = 4
    npoint = (N + stride - 1) // stride
    ncol = 128
    nrow = N // ncol
    tc = 128

    pts_t = jnp.transpose(points, (0, 2, 1))
    px2 = pts_t[:, 0, :].reshape(B, nrow, ncol)
    py2 = pts_t[:, 1, :].reshape(B, nrow, ncol)
    pz2 = pts_t[:, 2, :].reshape(B, nrow, ncol)

    sel = pl.pallas_call(
        functools.partial(_fps_kernel, nrow=nrow, ncol=ncol, npoint=npoint),
        grid=(B,),
        in_specs=[
            pl.BlockSpec((1, nrow, ncol), lambda b: (b, 0, 0)),
            pl.BlockSpec((1, nrow, ncol), lambda b: (b, 0, 0)),
            pl.BlockSpec((1, nrow, ncol), lambda b: (b, 0, 0)),
        ],
        out_specs=pl.BlockSpec((1, 3, npoint), lambda b: (b, 0, 0),
                               memory_space=pltpu.SMEM),
        out_shape=jax.ShapeDtypeStruct((B, 3, npoint), jnp.float32),
    )(px2, py2, pz2)

    sx = sel[:, 0, :, None]
    sy = sel[:, 1, :, None]
    sz = sel[:, 2, :, None]
    pxr = pts_t[:, 0:1, :]
    pyr = pts_t[:, 1:2, :]
    pzr = pts_t[:, 2:3, :]

    pooled = pl.pallas_call(
        functools.partial(_ball_pool_kernel, tc=tc, nrow=nrow, ncol=ncol),
        grid=(B, npoint // tc),
        in_specs=[
            pl.BlockSpec((1, tc, 1), lambda b, t: (b, t, 0)),
            pl.BlockSpec((1, tc, 1), lambda b, t: (b, t, 0)),
            pl.BlockSpec((1, tc, 1), lambda b, t: (b, t, 0)),
            pl.BlockSpec((1, 1, N), lambda b, t: (b, 0, 0)),
            pl.BlockSpec((1, 1, N), lambda b, t: (b, 0, 0)),
            pl.BlockSpec((1, 1, N), lambda b, t: (b, 0, 0)),
            pl.BlockSpec((1, N, C), lambda b, t: (b, 0, 0)),
        ],
        out_specs=pl.BlockSpec((1, tc, C), lambda b, t: (b, t, 0)),
        out_shape=jax.ShapeDtypeStruct((B, npoint, C), jnp.float32),
    )(sx, sy, sz, pxr, pyr, pzr, features)

    return (jnp.transpose(sel, (0, 2, 1)), pooled)


# speedup vs baseline: 262.9554x; 1.2244x over previous
import functools

import jax
import jax.numpy as jnp
from jax import lax
from jax.experimental import pallas as pl
from jax.experimental.pallas import tpu as pltpu

RADIUS2 = 0.2 * 0.2
NSAMPLE = 16


def _fps_kernel(px_ref, py_ref, pz_ref, sel_ref, *, nb, nrow, ncol, npoint):
    n = nrow * ncol
    px = [px_ref[b] for b in range(nb)]
    py = [py_ref[b] for b in range(nb)]
    pz = [pz_ref[b] for b in range(nb)]
    row_iota = lax.broadcasted_iota(jnp.int32, (nrow, ncol), 0)
    col_iota = lax.broadcasted_iota(jnp.int32, (nrow, ncol), 1)
    flat_iota = row_iota * ncol + col_iota
    lane_iota = lax.broadcasted_iota(jnp.int32, (1, ncol), 1)

    def body(i, carry):
        dists, far = carry
        new_dists, new_far = [], []
        for b in range(nb):
            r = far[b] // ncol
            c = far[b] % ncol
            lm = lane_iota == c
            cx = jnp.sum(jnp.where(lm, px_ref[b, pl.ds(r, 1), :], 0.0))
            cy = jnp.sum(jnp.where(lm, py_ref[b, pl.ds(r, 1), :], 0.0))
            cz = jnp.sum(jnp.where(lm, pz_ref[b, pl.ds(r, 1), :], 0.0))
            sel_ref[b, 0, i] = cx
            sel_ref[b, 1, i] = cy
            sel_ref[b, 2, i] = cz
            dx = px[b] - cx
            dy = py[b] - cy
            dz = pz[b] - cz
            d = dx * dx + dy * dy + dz * dz
            db = jnp.minimum(dists[b], d)
            m = jnp.max(db)
            fb = jnp.min(jnp.where(db == m, flat_iota, jnp.int32(n)))
            new_dists.append(db)
            new_far.append(fb)
        return tuple(new_dists), tuple(new_far)

    dists0 = tuple(jnp.full((nrow, ncol), 1e10, jnp.float32) for _ in range(nb))
    far0 = tuple(jnp.int32(0) for _ in range(nb))
    lax.fori_loop(0, npoint, body, (dists0, far0))


def _ball_pool_kernel(sx_ref, sy_ref, sz_ref, px_ref, py_ref, pz_ref,
                      feat_ref, out_ref, *, tc, nrow, ncol):
    n = nrow * ncol
    cx = sx_ref[0]
    cy = sy_ref[0]
    cz = sz_ref[0]
    pxr = px_ref[0]
    pyr = py_ref[0]
    pzr = pz_ref[0]
    dx = cx - pxr
    dy = cy - pyr
    dz = cz - pzr
    d2 = dx * dx + dy * dy + dz * dz
    w = (d2 < RADIUS2).astype(jnp.float32)
    w3 = w.reshape(tc, nrow, ncol)

    i0 = lax.broadcasted_iota(jnp.int32, (ncol, ncol), 0)
    i1 = lax.broadcasted_iota(jnp.int32, (ncol, ncol), 1)
    triu = (i0 <= i1).astype(jnp.float32)
    incl = lax.dot_general(w3, triu, (((2,), (0,)), ((), ())),
                           preferred_element_type=jnp.float32)
    blocktot = incl[:, :, ncol - 1]
    b0 = lax.broadcasted_iota(jnp.int32, (nrow, nrow), 0)
    b1 = lax.broadcasted_iota(jnp.int32, (nrow, nrow), 1)
    sup = (b0 < b1).astype(jnp.float32)
    offs = lax.dot_general(blocktot, sup, (((1,), (0,)), ((), ())),
                           preferred_element_type=jnp.float32)
    rank_ex = offs[:, :, None] + incl - w3

    w16 = w3 * (rank_ex < float(NSAMPLE)).astype(jnp.float32)
    first = w3 * (rank_ex == 0.0).astype(jnp.float32)
    total = offs[:, nrow - 1] + blocktot[:, nrow - 1]
    cnt16 = jnp.minimum(total, float(NSAMPLE))
    pw = w16 + first * (float(NSAMPLE) - cnt16)[:, None, None]
    zer = (lax.broadcasted_iota(jnp.int32, (nrow, ncol), 0) == 0) & (
        lax.broadcasted_iota(jnp.int32, (nrow, ncol), 1) == 0)
    pw = pw + (total == 0.0)[:, None, None] * zer[None].astype(jnp.float32) * float(NSAMPLE)

    pmat = pw.reshape(tc, n)
    pooled = lax.dot_general(pmat, feat_ref[0], (((1,), (0,)), ((), ())),
                             preferred_element_type=jnp.float32)
    out_ref[0] = pooled * (1.0 / NSAMPLE)


def kernel(points, features):
    B, N, C = features.shape
    stride = 4
    npoint = (N + stride - 1) // stride
    ncol = 128
    nrow = N // ncol
    tc = 128

    pts_t = jnp.transpose(points, (0, 2, 1))
    px2 = pts_t[:, 0, :].reshape(B, nrow, ncol)
    py2 = pts_t[:, 1, :].reshape(B, nrow, ncol)
    pz2 = pts_t[:, 2, :].reshape(B, nrow, ncol)

    sel = pl.pallas_call(
        functools.partial(_fps_kernel, nb=B, nrow=nrow, ncol=ncol,
                          npoint=npoint),
        grid=(1,),
        in_specs=[
            pl.BlockSpec((B, nrow, ncol), lambda g: (0, 0, 0)),
            pl.BlockSpec((B, nrow, ncol), lambda g: (0, 0, 0)),
            pl.BlockSpec((B, nrow, ncol), lambda g: (0, 0, 0)),
        ],
        out_specs=pl.BlockSpec((B, 3, npoint), lambda g: (0, 0, 0),
                               memory_space=pltpu.SMEM),
        out_shape=jax.ShapeDtypeStruct((B, 3, npoint), jnp.float32),
    )(px2, py2, pz2)

    sx = sel[:, 0, :, None]
    sy = sel[:, 1, :, None]
    sz = sel[:, 2, :, None]
    pxr = pts_t[:, 0:1, :]
    pyr = pts_t[:, 1:2, :]
    pzr = pts_t[:, 2:3, :]

    pooled = pl.pallas_call(
        functools.partial(_ball_pool_kernel, tc=tc, nrow=nrow, ncol=ncol),
        grid=(B, npoint // tc),
        in_specs=[
            pl.BlockSpec((1, tc, 1), lambda b, t: (b, t, 0)),
            pl.BlockSpec((1, tc, 1), lambda b, t: (b, t, 0)),
            pl.BlockSpec((1, tc, 1), lambda b, t: (b, t, 0)),
            pl.BlockSpec((1, 1, N), lambda b, t: (b, 0, 0)),
            pl.BlockSpec((1, 1, N), lambda b, t: (b, 0, 0)),
            pl.BlockSpec((1, 1, N), lambda b, t: (b, 0, 0)),
            pl.BlockSpec((1, N, C), lambda b, t: (b, 0, 0)),
        ],
        out_specs=pl.BlockSpec((1, tc, C), lambda b, t: (b, t, 0)),
        out_shape=jax.ShapeDtypeStruct((B, npoint, C), jnp.float32),
    )(sx, sy, sz, pxr, pyr, pzr, features)

    return (jnp.transpose(sel, (0, 2, 1)), pooled)


# speedup vs baseline: 263.1938x; 1.0009x over previous
import functools

import jax
import jax.numpy as jnp
from jax import lax
from jax.experimental import pallas as pl
from jax.experimental.pallas import tpu as pltpu

RADIUS2 = 0.2 * 0.2
NSAMPLE = 16


def _fps_kernel(px_ref, py_ref, pz_ref, sel_ref, *dist_refs, nb, nrow, ncol,
                npoint):
    n = nrow * ncol
    row_iota = lax.broadcasted_iota(jnp.int32, (nrow, ncol), 0)
    col_iota = lax.broadcasted_iota(jnp.int32, (nrow, ncol), 1)
    flat_iota = row_iota * ncol + col_iota
    lane_iota = lax.broadcasted_iota(jnp.int32, (1, ncol), 1)

    for b in range(nb):
        dist_refs[b][...] = jnp.full((nrow, ncol), 1e10, jnp.float32)

    def body(i, far):
        new_far = []
        for b in range(nb):
            r = far[b] // ncol
            c = far[b] % ncol
            lm = lane_iota == c
            cx = jnp.sum(jnp.where(lm, px_ref[b, pl.ds(r, 1), :], 0.0))
            cy = jnp.sum(jnp.where(lm, py_ref[b, pl.ds(r, 1), :], 0.0))
            cz = jnp.sum(jnp.where(lm, pz_ref[b, pl.ds(r, 1), :], 0.0))
            sel_ref[b, 0, i] = cx
            sel_ref[b, 1, i] = cy
            sel_ref[b, 2, i] = cz
            dx = px_ref[b] - cx
            dy = py_ref[b] - cy
            dz = pz_ref[b] - cz
            d = dx * dx + dy * dy + dz * dz
            db = jnp.minimum(dist_refs[b][...], d)
            dist_refs[b][...] = db
            m = jnp.max(db)
            fb = jnp.min(jnp.where(db == m, flat_iota, jnp.int32(n)))
            new_far.append(fb)
        return tuple(new_far)

    lax.fori_loop(0, npoint, body, tuple(jnp.int32(0) for _ in range(nb)))


def _ball_pool_kernel(sx_ref, sy_ref, sz_ref, px_ref, py_ref, pz_ref,
                      feat_ref, out_ref, *, tc, nrow, ncol):
    n = nrow * ncol
    cx = sx_ref[0]
    cy = sy_ref[0]
    cz = sz_ref[0]
    pxr = px_ref[0]
    pyr = py_ref[0]
    pzr = pz_ref[0]
    dx = cx - pxr
    dy = cy - pyr
    dz = cz - pzr
    d2 = dx * dx + dy * dy + dz * dz
    w = (d2 < RADIUS2).astype(jnp.float32)
    w3 = w.reshape(tc, nrow, ncol)

    i0 = lax.broadcasted_iota(jnp.int32, (ncol, ncol), 0)
    i1 = lax.broadcasted_iota(jnp.int32, (ncol, ncol), 1)
    triu = (i0 <= i1).astype(jnp.float32)
    incl = lax.dot_general(w3, triu, (((2,), (0,)), ((), ())),
                           preferred_element_type=jnp.float32)
    blocktot = incl[:, :, ncol - 1]
    b0 = lax.broadcasted_iota(jnp.int32, (nrow, nrow), 0)
    b1 = lax.broadcasted_iota(jnp.int32, (nrow, nrow), 1)
    sup = (b0 < b1).astype(jnp.float32)
    offs = lax.dot_general(blocktot, sup, (((1,), (0,)), ((), ())),
                           preferred_element_type=jnp.float32)
    rank_ex = offs[:, :, None] + incl - w3

    w16 = w3 * (rank_ex < float(NSAMPLE)).astype(jnp.float32)
    first = w3 * (rank_ex == 0.0).astype(jnp.float32)
    total = offs[:, nrow - 1] + blocktot[:, nrow - 1]
    cnt16 = jnp.minimum(total, float(NSAMPLE))
    pw = w16 + first * (float(NSAMPLE) - cnt16)[:, None, None]
    zer = (lax.broadcasted_iota(jnp.int32, (nrow, ncol), 0) == 0) & (
        lax.broadcasted_iota(jnp.int32, (nrow, ncol), 1) == 0)
    pw = pw + (total == 0.0)[:, None, None] * zer[None].astype(jnp.float32) * float(NSAMPLE)

    pmat = pw.reshape(tc, n)
    pooled = lax.dot_general(pmat, feat_ref[0], (((1,), (0,)), ((), ())),
                             preferred_element_type=jnp.float32)
    out_ref[0] = pooled * (1.0 / NSAMPLE)


def kernel(points, features):
    B, N, C = features.shape
    stride = 4
    npoint = (N + stride - 1) // stride
    ncol = 128
    nrow = N // ncol
    tc = 128

    pts_t = jnp.transpose(points, (0, 2, 1))
    px2 = pts_t[:, 0, :].reshape(B, nrow, ncol)
    py2 = pts_t[:, 1, :].reshape(B, nrow, ncol)
    pz2 = pts_t[:, 2, :].reshape(B, nrow, ncol)

    sel = pl.pallas_call(
        functools.partial(_fps_kernel, nb=B, nrow=nrow, ncol=ncol,
                          npoint=npoint),
        grid=(1,),
        in_specs=[
            pl.BlockSpec((B, nrow, ncol), lambda g: (0, 0, 0)),
            pl.BlockSpec((B, nrow, ncol), lambda g: (0, 0, 0)),
            pl.BlockSpec((B, nrow, ncol), lambda g: (0, 0, 0)),
        ],
        out_specs=pl.BlockSpec((B, 3, npoint), lambda g: (0, 0, 0),
                               memory_space=pltpu.SMEM),
        out_shape=jax.ShapeDtypeStruct((B, 3, npoint), jnp.float32),
        scratch_shapes=[pltpu.VMEM((nrow, ncol), jnp.float32)
                        for _ in range(B)],
    )(px2, py2, pz2)

    sx = sel[:, 0, :, None]
    sy = sel[:, 1, :, None]
    sz = sel[:, 2, :, None]
    pxr = pts_t[:, 0:1, :]
    pyr = pts_t[:, 1:2, :]
    pzr = pts_t[:, 2:3, :]

    pooled = pl.pallas_call(
        functools.partial(_ball_pool_kernel, tc=tc, nrow=nrow, ncol=ncol),
        grid=(B, npoint // tc),
        in_specs=[
            pl.BlockSpec((1, tc, 1), lambda b, t: (b, t, 0)),
            pl.BlockSpec((1, tc, 1), lambda b, t: (b, t, 0)),
            pl.BlockSpec((1, tc, 1), lambda b, t: (b, t, 0)),
            pl.BlockSpec((1, 1, N), lambda b, t: (b, 0, 0)),
            pl.BlockSpec((1, 1, N), lambda b, t: (b, 0, 0)),
            pl.BlockSpec((1, 1, N), lambda b, t: (b, 0, 0)),
            pl.BlockSpec((1, N, C), lambda b, t: (b, 0, 0)),
        ],
        out_specs=pl.BlockSpec((1, tc, C), lambda b, t: (b, t, 0)),
        out_shape=jax.ShapeDtypeStruct((B, npoint, C), jnp.float32),
    )(sx, sy, sz, pxr, pyr, pzr, features)

    return (jnp.transpose(sel, (0, 2, 1)), pooled)


# speedup vs baseline: 377.4855x; 1.4342x over previous
import functools

import jax
import jax.numpy as jnp
from jax import lax
from jax.experimental import pallas as pl
from jax.experimental.pallas import tpu as pltpu

RADIUS2 = 0.2 * 0.2
NSAMPLE = 16


def _fps_kernel(px_ref, py_ref, pz_ref, sel_ref, sx_ref, sy_ref, sz_ref,
                sem, *dist_refs, nb, nrow, ncol, npoint):
    n = nrow * ncol
    row_iota = lax.broadcasted_iota(jnp.int32, (nrow, ncol), 0)
    col_iota = lax.broadcasted_iota(jnp.int32, (nrow, ncol), 1)
    flat_iota = row_iota * ncol + col_iota

    cpx = pltpu.make_async_copy(px_ref, sx_ref, sem)
    cpx.start()
    cpy = pltpu.make_async_copy(py_ref, sy_ref, sem)
    cpy.start()
    cpz = pltpu.make_async_copy(pz_ref, sz_ref, sem)
    cpz.start()
    for b in range(nb):
        dist_refs[b][...] = jnp.full((nrow, ncol), 1e10, jnp.float32)
    cpx.wait()
    cpy.wait()
    cpz.wait()

    def body(i, far):
        new_far = []
        for b in range(nb):
            r = far[b] // ncol
            c = far[b] % ncol
            cx = sx_ref[b, r, c]
            cy = sy_ref[b, r, c]
            cz = sz_ref[b, r, c]
            sel_ref[b, 0, i] = cx
            sel_ref[b, 1, i] = cy
            sel_ref[b, 2, i] = cz
            dx = px_ref[b] - cx
            dy = py_ref[b] - cy
            dz = pz_ref[b] - cz
            d = dx * dx + dy * dy + dz * dz
            db = jnp.minimum(dist_refs[b][...], d)
            dist_refs[b][...] = db
            m = jnp.max(db)
            fb = jnp.min(jnp.where(db == m, flat_iota, jnp.int32(n)))
            new_far.append(fb)
        return tuple(new_far)

    lax.fori_loop(0, npoint, body, tuple(jnp.int32(0) for _ in range(nb)))


def _ball_pool_kernel(sx_ref, sy_ref, sz_ref, px_ref, py_ref, pz_ref,
                      feat_ref, out_ref, *, tc, nrow, ncol):
    n = nrow * ncol
    cx = sx_ref[0]
    cy = sy_ref[0]
    cz = sz_ref[0]
    pxr = px_ref[0]
    pyr = py_ref[0]
    pzr = pz_ref[0]
    dx = cx - pxr
    dy = cy - pyr
    dz = cz - pzr
    d2 = dx * dx + dy * dy + dz * dz
    w = (d2 < RADIUS2).astype(jnp.float32)
    w3 = w.reshape(tc, nrow, ncol)

    i0 = lax.broadcasted_iota(jnp.int32, (ncol, ncol), 0)
    i1 = lax.broadcasted_iota(jnp.int32, (ncol, ncol), 1)
    triu = (i0 <= i1).astype(jnp.float32)
    incl = lax.dot_general(w3, triu, (((2,), (0,)), ((), ())),
                           preferred_element_type=jnp.float32)
    blocktot = incl[:, :, ncol - 1]
    b0 = lax.broadcasted_iota(jnp.int32, (nrow, nrow), 0)
    b1 = lax.broadcasted_iota(jnp.int32, (nrow, nrow), 1)
    sup = (b0 < b1).astype(jnp.float32)
    offs = lax.dot_general(blocktot, sup, (((1,), (0,)), ((), ())),
                           preferred_element_type=jnp.float32)
    rank_ex = offs[:, :, None] + incl - w3

    w16 = w3 * (rank_ex < float(NSAMPLE)).astype(jnp.float32)
    first = w3 * (rank_ex == 0.0).astype(jnp.float32)
    total = offs[:, nrow - 1] + blocktot[:, nrow - 1]
    cnt16 = jnp.minimum(total, float(NSAMPLE))
    pw = w16 + first * (float(NSAMPLE) - cnt16)[:, None, None]
    zer = (lax.broadcasted_iota(jnp.int32, (nrow, ncol), 0) == 0) & (
        lax.broadcasted_iota(jnp.int32, (nrow, ncol), 1) == 0)
    pw = pw + (total == 0.0)[:, None, None] * zer[None].astype(jnp.float32) * float(NSAMPLE)

    pmat = pw.reshape(tc, n)
    pooled = lax.dot_general(pmat, feat_ref[0], (((1,), (0,)), ((), ())),
                             preferred_element_type=jnp.float32)
    out_ref[0] = pooled * (1.0 / NSAMPLE)


def kernel(points, features):
    B, N, C = features.shape
    stride = 4
    npoint = (N + stride - 1) // stride
    ncol = 128
    nrow = N // ncol
    tc = 128

    pts_t = jnp.transpose(points, (0, 2, 1))
    px2 = pts_t[:, 0, :].reshape(B, nrow, ncol)
    py2 = pts_t[:, 1, :].reshape(B, nrow, ncol)
    pz2 = pts_t[:, 2, :].reshape(B, nrow, ncol)

    sel = pl.pallas_call(
        functools.partial(_fps_kernel, nb=B, nrow=nrow, ncol=ncol,
                          npoint=npoint),
        grid=(1,),
        in_specs=[
            pl.BlockSpec((B, nrow, ncol), lambda g: (0, 0, 0)),
            pl.BlockSpec((B, nrow, ncol), lambda g: (0, 0, 0)),
            pl.BlockSpec((B, nrow, ncol), lambda g: (0, 0, 0)),
        ],
        out_specs=pl.BlockSpec((B, 3, npoint), lambda g: (0, 0, 0),
                               memory_space=pltpu.SMEM),
        out_shape=jax.ShapeDtypeStruct((B, 3, npoint), jnp.float32),
        scratch_shapes=[pltpu.SMEM((B, nrow, ncol), jnp.float32),
                        pltpu.SMEM((B, nrow, ncol), jnp.float32),
                        pltpu.SMEM((B, nrow, ncol), jnp.float32),
                        pltpu.SemaphoreType.DMA]
                       + [pltpu.VMEM((nrow, ncol), jnp.float32)
                          for _ in range(B)],
    )(px2, py2, pz2)

    sx = sel[:, 0, :, None]
    sy = sel[:, 1, :, None]
    sz = sel[:, 2, :, None]
    pxr = pts_t[:, 0:1, :]
    pyr = pts_t[:, 1:2, :]
    pzr = pts_t[:, 2:3, :]

    pooled = pl.pallas_call(
        functools.partial(_ball_pool_kernel, tc=tc, nrow=nrow, ncol=ncol),
        grid=(B, npoint // tc),
        in_specs=[
            pl.BlockSpec((1, tc, 1), lambda b, t: (b, t, 0)),
            pl.BlockSpec((1, tc, 1), lambda b, t: (b, t, 0)),
            pl.BlockSpec((1, tc, 1), lambda b, t: (b, t, 0)),
            pl.BlockSpec((1, 1, N), lambda b, t: (b, 0, 0)),
            pl.BlockSpec((1, 1, N), lambda b, t: (b, 0, 0)),
            pl.BlockSpec((1, 1, N), lambda b, t: (b, 0, 0)),
            pl.BlockSpec((1, N, C), lambda b, t: (b, 0, 0)),
        ],
        out_specs=pl.BlockSpec((1, tc, C), lambda b, t: (b, t, 0)),
        out_shape=jax.ShapeDtypeStruct((B, npoint, C), jnp.float32),
    )(sx, sy, sz, pxr, pyr, pzr, features)

    return (jnp.transpose(sel, (0, 2, 1)), pooled)
